# baseline (speedup 1.0000x reference)
import functools

import jax
import jax.numpy as jnp
from jax import lax
from jax.experimental import pallas as pl
from jax.experimental.pallas import tpu as pltpu
from jax.experimental.pallas import tpu_sc as plsc

C = 128
RES = 256
NC = 2
NS = 16
NW = NC * NS
CB = 64
NCHUNK = 50
PW = CB * NCHUNK
N_PAD = NW * PW


def _tri_body(t0, t1, t2, m0h, m1h, m2h, outh,
              m0v, m1v, m2v, idxv, wv, taps, outv0, outv1,
              sem0, sem1, sem2, osem0, osem1):
    wid = lax.axis_index("c") * NS + lax.axis_index("s")
    base = wid * PW

    pltpu.sync_copy(m0h.at[pl.ds(base, PW)], m0v)
    pltpu.sync_copy(m1h.at[pl.ds(base, PW)], m1v)
    pltpu.sync_copy(m2h.at[pl.ds(base, PW)], m2v)

    tables = (t0, t1, t2)
    sems = (sem0, sem1, sem2)
    outvs = (outv0, outv1)
    osems = (osem0, osem1)

    def compute_idx(ci, half):
        for g in range(CB // 16):
            s = ci * CB + g * 16
            mm0 = jnp.clip(m0v[pl.ds(s, 16)], 0.0, 1.0 - 1e-6) * RES
            mm1 = jnp.clip(m1v[pl.ds(s, 16)], 0.0, 1.0 - 1e-6) * RES
            mm2 = jnp.clip(m2v[pl.ds(s, 16)], 0.0, 1.0 - 1e-6) * RES
            proj = ((mm0, mm1), (mm1, mm2), (mm2, mm0))
            for k in range(3):
                px, py = proj[k]
                xi = px.astype(jnp.int32)
                yi = py.astype(jnp.int32)
                wx = px - xi.astype(jnp.float32)
                wy = py - yi.astype(jnp.float32)
                xok = xi < RES - 1
                yok = yi < RES - 1
                xs = jnp.where(xok, 1, 0)
                ys = jnp.where(yok, RES, 0)
                wx1 = jnp.where(xok, wx, 0.0)
                wy1 = jnp.where(yok, wy, 0.0)
                i00 = yi * RES + xi
                row = half * 3 + k
                idxv[row, pl.ds(0 * CB + g * 16, 16)] = i00
                idxv[row, pl.ds(1 * CB + g * 16, 16)] = i00 + xs
                idxv[row, pl.ds(2 * CB + g * 16, 16)] = i00 + ys
                idxv[row, pl.ds(3 * CB + g * 16, 16)] = i00 + xs + ys
                wrow = (half * 12) + 4 * k
                gs = pl.ds(g * 16, 16)
                wv[wrow + 0, gs] = (1.0 - wx) * (1.0 - wy)
                wv[wrow + 1, gs] = wx1 * (1.0 - wy)
                wv[wrow + 2, gs] = (1.0 - wx) * wy1
                wv[wrow + 3, gs] = wx1 * wy1

    def fire(k, half):
        for t in range(4):
            pltpu.async_copy(
                tables[k].at[idxv.at[half * 3 + k, pl.ds(t * CB, CB)]],
                taps.at[pl.ds((4 * k + t) * CB, CB)], sems[k])

    def drain(k, half):
        for t in range(4):
            pltpu.make_async_copy(
                tables[k].at[idxv.at[half * 3 + k, pl.ds(t * CB, CB)]],
                taps.at[pl.ds((4 * k + t) * CB, CB)], sems[k]).wait()

    def combine_plane(k, half, par):
        ov = outvs[par]
        for g in range(CB // 16):
            rvec = lax.iota(jnp.int32, 16) + g * 16
            rows = [rvec + (4 * k + t) * CB for t in range(4)]
            ws = [wv[half * 12 + 4 * k + t, pl.ds(g * 16, 16)]
                  for t in range(4)]

            @plsc.parallel_loop(0, C, unroll=1)
            def cbody(c, rows=rows, ws=ws):
                cvec = (jnp.full((16,), c, jnp.int32)
                        + lax.iota(jnp.int32, 16)) & (C - 1)
                acc = None
                for t in range(4):
                    term = ws[t] * plsc.load_gather(taps, [rows[t], cvec])
                    acc = term if acc is None else acc + term
                if k > 0:
                    acc = acc * plsc.load_gather(ov, [rvec, cvec])
                plsc.store_scatter(ov, [rvec, cvec], acc)

    def out_fire(ci, par):
        pltpu.async_copy(outvs[par], outh.at[pl.ds(base + ci * CB, CB)],
                         osems[par])

    def out_wait(par):
        pltpu.make_async_copy(outvs[par], outh.at[pl.ds(base, CB)],
                              osems[par]).wait()

    compute_idx(0, 0)
    for k in range(3):
        fire(k, 0)

    def step_body(s, carry):
        c0 = 2 * s
        compute_idx(c0 + 1, 1)

        @pl.when(s > 0)
        def _():
            out_wait(0)

        for k in range(3):
            drain(k, 0)
            combine_plane(k, 0, 0)
            fire(k, 1)
        out_fire(c0, 0)

        @pl.when(s < NCHUNK // 2 - 1)
        def _():
            compute_idx(c0 + 2, 0)

        @pl.when(s > 0)
        def _():
            out_wait(1)

        for k in range(3):
            drain(k, 1)
            combine_plane(k, 1, 1)

            @pl.when(s < NCHUNK // 2 - 1)
            def _(k=k):
                fire(k, 0)

        out_fire(c0 + 1, 1)
        return carry

    lax.fori_loop(0, NCHUNK // 2, step_body, 0)
    out_wait(0)
    out_wait(1)


_tri = pl.kernel(
    _tri_body,
    out_type=jax.ShapeDtypeStruct((N_PAD, C), jnp.float32),
    mesh=plsc.VectorSubcoreMesh(core_axis_name="c", subcore_axis_name="s"),
    compiler_params=pltpu.CompilerParams(needs_layout_passes=False,
                                         disable_bounds_checks=True),
    scratch_types=[
        pltpu.VMEM((PW,), jnp.float32),
        pltpu.VMEM((PW,), jnp.float32),
        pltpu.VMEM((PW,), jnp.float32),
        pltpu.VMEM((6, 4 * CB), jnp.int32),
        pltpu.VMEM((24, CB), jnp.float32),
        pltpu.VMEM((12 * CB, C), jnp.float32),
        pltpu.VMEM((CB, C), jnp.float32),
        pltpu.VMEM((CB, C), jnp.float32),
        pltpu.SemaphoreType.DMA,
        pltpu.SemaphoreType.DMA,
        pltpu.SemaphoreType.DMA,
        pltpu.SemaphoreType.DMA,
        pltpu.SemaphoreType.DMA,
    ],
)


def kernel(mu, P0, P1, P2):
    n = mu.shape[0]
    tb0 = jnp.transpose(P0.reshape(C, RES * RES))
    tb1 = jnp.transpose(P1.reshape(C, RES * RES))
    tb2 = jnp.transpose(P2.reshape(C, RES * RES))
    mt = jnp.pad(mu, ((0, N_PAD - n), (0, 0))).T
    out = _tri(tb0, tb1, tb2, mt[0], mt[1], mt[2])
    return out[:n]
